# baseline (speedup 1.0000x reference)
import jax
import jax.numpy as jnp
from jax import lax
from jax.experimental import pallas as pl
from jax.experimental.pallas import tpu as pltpu
from jax.experimental.pallas import tpu_sc as plsc

N = 10000
D = 128
DH = D // 2
E = 320000
E_TOT = E + N
N_OUT = 10112
RT_OUT = N_OUT // 16
OUT_CHUNKS = (128, 128, 128, 128, 120)
N_DEN = 10240
RT_DEN = N_DEN // 16
C = 512
NJ = C // 128
CH = 42
PER_W = C * CH
E_PAD = PER_W * 16


def _tc_prep(x_ref, w_ref, as_ref, ad_ref, ha_ref, hb_ref, av_ref, bv_ref,
             m_ref):
    h = jnp.dot(x_ref[...], w_ref[...], preferred_element_type=jnp.float32)
    ha_ref[...] = h[:, :DH]
    hb_ref[...] = h[:, DH:]
    a1 = jnp.sum(h * as_ref[...], axis=1, keepdims=True)
    a2 = jnp.sum(h * ad_ref[...], axis=1, keepdims=True)
    av_ref[...] = a1
    bv_ref[...] = a2
    ms = jnp.max(a1) + jnp.max(a2)
    m_ref[...] = jnp.full((1, 1), jnp.where(ms >= 0, ms, ms * 0.2))


def _tc_finish(p0_ref, p1_ref, d_ref, b_ref, o_ref):
    den = d_ref[...]
    o_ref[...] = (jnp.concatenate([p0_ref[...], p1_ref[...]], axis=1) / den
                  + b_ref[...])


def _sc_edges(ha_hbm, hb_hbm, asrc_hbm, adst_hbm, src_hbm, dst_hbm, m_hbm,
              outp_hbm, outd_hbm,
              asv, adv, idx_s0, idx_s1, idx_d0, idx_d1, idx_d2,
              wv0, wv1, rows0, rows1, z640, mv,
              shared_out, shared_den, sem_i, sem_g, sem_s):
    c = lax.axis_index("c")
    s = lax.axis_index("s")
    zero16 = jnp.zeros((16,), jnp.float32)
    idx_s = (idx_s0, idx_s1)
    idx_d = (idx_d0, idx_d1, idx_d2)
    rows = (rows0, rows1)
    wv = (wv0, wv1)

    for i in range(RT_DEN // 16):
        z640[pl.ds(i * 16, 16)] = zero16

    def _zrow(i, _):
        for j in range(DH // 16):
            rows0[i, pl.ds(j * 16, 16)] = zero16
        return 0
    lax.fori_loop(0, 128, _zrow, 0)

    rowo = s * RT_OUT
    rowd = s * RT_DEN
    ko = 0
    for ck in OUT_CHUNKS:
        pltpu.sync_copy(rows0.at[pl.ds(0, ck)],
                        shared_out.at[pl.ds(rowo + ko, ck)])
        ko += ck
    pltpu.sync_copy(z640, shared_den.at[pl.ds(rowd, RT_DEN)])

    pltpu.sync_copy(asrc_hbm, asv)
    pltpu.sync_copy(adst_hbm, adv)
    pltpu.sync_copy(m_hbm, mv)
    mvec = mv[...]
    plsc.subcore_barrier()

    base128 = s * (PER_W // 128)

    def _issue_idx(ch, sb, db):
        b128 = base128 + ch * NJ
        pltpu.async_copy(src_hbm.at[pl.ds(b128, NJ)], idx_s[sb], sem_i)
        pltpu.async_copy(dst_hbm.at[pl.ds(b128, NJ)], idx_d[db], sem_i)

    def _drain_idx(sb, db):
        pltpu.make_async_copy(src_hbm.at[pl.ds(base128, NJ)], idx_s[sb],
                              sem_i).wait()
        pltpu.make_async_copy(dst_hbm.at[pl.ds(base128, NJ)], idx_d[db],
                              sem_i).wait()

    def _fire_gathers(sb, rb):
        for j in range(NJ):
            @pl.when(c == 0)
            def _():
                pltpu.async_copy(ha_hbm.at[idx_s[sb].at[j]],
                                 rows[rb].at[pl.ds(j * 128, 128)], sem_g)

            @pl.when(c == 1)
            def _():
                pltpu.async_copy(hb_hbm.at[idx_s[sb].at[j]],
                                 rows[rb].at[pl.ds(j * 128, 128)], sem_g)

    def _drain_gathers(sb, rb):
        for j in range(NJ):
            pltpu.make_async_copy(ha_hbm.at[idx_s[sb].at[j]],
                                  rows[rb].at[pl.ds(j * 128, 128)],
                                  sem_g).wait()

    def _drain_scatters(rb, db):
        for j in range(NJ):
            pltpu.make_async_copy(rows[rb].at[pl.ds(j * 128, 128)],
                                  shared_out.at[idx_d[db].at[j]],
                                  sem_s).wait()
            pltpu.make_async_copy(wv[rb].at[pl.ds(j * 128, 128)],
                                  shared_den.at[idx_d[db].at[j]],
                                  sem_s).wait()

    def _chunk(t, g, k):
        rb, rbn = k % 2, (k + 1) % 2
        db, dbp = k % 3, (k + 2) % 3
        _drain_gathers(rb, rb)
        ebase = (base128 + g * NJ) * 128
        for i in range(C // 16):
            s16 = idx_s[rb][i // 8, pl.ds((i % 8) * 16, 16)]
            d16 = idx_d[db][i // 8, pl.ds((i % 8) * 16, 16)]
            e = plsc.load_gather(asv, [s16]) + plsc.load_gather(adv, [d16])
            e = jnp.where(e >= 0, e, e * 0.2)
            w = jnp.exp(e - mvec)
            eidx = ebase + i * 16 + lax.iota(jnp.int32, 16)
            w = jnp.where(eidx < E_TOT, w, 0.0)
            wv[rb][pl.ds(i * 16, 16)] = w
        _drain_idx(rbn, (k + 1) % 3)
        if k == 0:
            @pl.when(t >= 1)
            def _():
                _drain_scatters(rbn, dbp)
        else:
            _drain_scatters(rbn, dbp)
        _fire_gathers(rbn, rbn)
        _issue_idx(jnp.minimum(g + 2, CH - 1), rb, dbp)

        def _srow(gg, _):
            w16 = wv[rb][pl.ds(gg * 16, 16)]
            for l in range(16):
                wf = jnp.full((16,), w16[l])
                r = gg * 16 + l
                for j in range(DH // 16):
                    rows[rb][r, pl.ds(j * 16, 16)] = (
                        rows[rb][r, pl.ds(j * 16, 16)] * wf)
            return 0
        lax.fori_loop(0, C // 16, _srow, 0)

        for j in range(NJ):
            pltpu.async_copy(rows[rb].at[pl.ds(j * 128, 128)],
                             shared_out.at[idx_d[db].at[j]], sem_s, add=True)
            pltpu.async_copy(wv[rb].at[pl.ds(j * 128, 128)],
                             shared_den.at[idx_d[db].at[j]], sem_s, add=True)

    _issue_idx(0, 0, 0)
    _drain_idx(0, 0)
    _fire_gathers(0, 0)
    _issue_idx(1, 1, 1)

    def _step(t, _):
        for k in range(6):
            _chunk(t, 6 * t + k, k)
        return 0
    lax.fori_loop(0, CH // 6, _step, 0)

    _drain_scatters(1, (CH - 1) % 3)
    _drain_gathers(0, 0)
    _drain_idx(0, 0)

    plsc.subcore_barrier()

    offo = c * N_OUT + rowo
    ko = 0
    for k, ck in enumerate(OUT_CHUNKS):
        buf = rows[k % 2].at[pl.ds(0, ck)]
        pltpu.sync_copy(shared_out.at[pl.ds(rowo + ko, ck)], buf)
        pltpu.sync_copy(buf, outp_hbm.at[pl.ds(offo + ko, ck)])
        ko += ck
    pltpu.sync_copy(shared_den.at[pl.ds(rowd, RT_DEN)], z640)
    pltpu.sync_copy(z640, outd_hbm.at[pl.ds(c * N_DEN + rowd, RT_DEN)])


def kernel(x, edge_index, W, att_src, att_dst, bias):
    f32 = jnp.float32
    ha, hb, av, bv, m = pl.pallas_call(
        _tc_prep,
        out_shape=(
            jax.ShapeDtypeStruct((N, DH), f32),
            jax.ShapeDtypeStruct((N, DH), f32),
            jax.ShapeDtypeStruct((N, 1), f32),
            jax.ShapeDtypeStruct((N, 1), f32),
            jax.ShapeDtypeStruct((1, 1), f32),
        ),
    )(x, W, att_src.reshape(1, D), att_dst.reshape(1, D))

    loop = jnp.arange(N, dtype=jnp.int32)
    pad = jnp.zeros((E_PAD - E_TOT,), jnp.int32)
    src2d = jnp.concatenate([edge_index[0], loop, pad]).reshape(E_PAD // 128, 128)
    dst2d = jnp.concatenate([edge_index[1], loop, pad]).reshape(E_PAD // 128, 128)
    m16 = jnp.broadcast_to(m.reshape(1), (16,))

    sc = pl.kernel(
        _sc_edges,
        out_type=(
            jax.ShapeDtypeStruct((2 * N_OUT, DH), f32),
            jax.ShapeDtypeStruct((2 * N_DEN,), f32),
        ),
        mesh=plsc.VectorSubcoreMesh(core_axis_name="c", subcore_axis_name="s"),
        compiler_params=pltpu.CompilerParams(
            needs_layout_passes=False, use_tc_tiling_on_sc=False),
        scratch_types=(
            pltpu.VMEM((N,), f32),
            pltpu.VMEM((N,), f32),
            pltpu.VMEM((NJ, 128), jnp.int32),
            pltpu.VMEM((NJ, 128), jnp.int32),
            pltpu.VMEM((NJ, 128), jnp.int32),
            pltpu.VMEM((NJ, 128), jnp.int32),
            pltpu.VMEM((NJ, 128), jnp.int32),
            pltpu.VMEM((C,), f32),
            pltpu.VMEM((C,), f32),
            pltpu.VMEM((C, DH), f32),
            pltpu.VMEM((C, DH), f32),
            pltpu.VMEM((RT_DEN,), f32),
            pltpu.VMEM((16,), f32),
            pltpu.VMEM_SHARED((N_OUT, DH), f32),
            pltpu.VMEM_SHARED((N_DEN,), f32),
            pltpu.SemaphoreType.DMA,
            pltpu.SemaphoreType.DMA,
            pltpu.SemaphoreType.DMA,
        ),
    )
    outp, outd = sc(ha, hb, av.reshape(N), bv.reshape(N), src2d, dst2d, m16)

    p = outp.reshape(2, N_OUT, DH)
    d = outd.reshape(2, N_DEN)
    out = pl.pallas_call(
        _tc_finish,
        out_shape=jax.ShapeDtypeStruct((N, D), f32),
    )(p[0, :N], p[1, :N], d[0, :N, None], bias.reshape(1, D))
    return out


# speedup vs baseline: 1.1239x; 1.1239x over previous
import jax
import jax.numpy as jnp
from jax import lax
from jax.experimental import pallas as pl
from jax.experimental.pallas import tpu as pltpu
from jax.experimental.pallas import tpu_sc as plsc

N = 10000
D = 128
DH = D // 2
E = 320000
E_TOT = E + N
N_OUT = 10112
RT_OUT = N_OUT // 16
OUT_CHUNKS = (128, 128, 128, 128, 120)
N_DEN = 10240
RT_DEN = N_DEN // 16
C = 512
NJ = C // 128
CH = 42
PER_W = C * CH
E_PAD = PER_W * 16


def _tc_prep(x_ref, w_ref, as_ref, ad_ref, ha_ref, hb_ref, av_ref, bv_ref,
             m_ref):
    h = jnp.dot(x_ref[...], w_ref[...], preferred_element_type=jnp.float32)
    ha_ref[...] = h[:, :DH]
    hb_ref[...] = h[:, DH:]
    a1 = jnp.sum(h * as_ref[...], axis=1, keepdims=True)
    a2 = jnp.sum(h * ad_ref[...], axis=1, keepdims=True)
    av_ref[...] = a1
    bv_ref[...] = a2
    ms = jnp.max(a1) + jnp.max(a2)
    m_ref[...] = jnp.full((1, 1), jnp.where(ms >= 0, ms, ms * 0.2))


def _tc_finish(p0_ref, p1_ref, d_ref, b_ref, o_ref):
    den = d_ref[...]
    o_ref[...] = (jnp.concatenate([p0_ref[...], p1_ref[...]], axis=1) / den
                  + b_ref[...])


def _sc_edges(ha_hbm, hb_hbm, asrc_hbm, adst_hbm, src_hbm, dst_hbm, m_hbm,
              outp_hbm, outd_hbm,
              asv, adv, idx_s0, idx_s1, idx_d0, idx_d1, idx_d2,
              wv0, wv1, rows0, rows1, z640, mv,
              shared_out, shared_den, sem_i, sem_g, sem_s):
    c = lax.axis_index("c")
    s = lax.axis_index("s")
    zero16 = jnp.zeros((16,), jnp.float32)
    idx_s = (idx_s0, idx_s1)
    idx_d = (idx_d0, idx_d1, idx_d2)
    rows = (rows0, rows1)
    wv = (wv0, wv1)

    for i in range(RT_DEN // 16):
        z640[pl.ds(i * 16, 16)] = zero16

    def _zrow(i, _):
        for j in range(DH // 16):
            rows0[i, pl.ds(j * 16, 16)] = zero16
        return 0
    lax.fori_loop(0, 128, _zrow, 0)

    rowo = s * RT_OUT
    rowd = s * RT_DEN
    ko = 0
    for ck in OUT_CHUNKS:
        pltpu.sync_copy(rows0.at[pl.ds(0, ck)],
                        shared_out.at[pl.ds(rowo + ko, ck)])
        ko += ck
    pltpu.sync_copy(z640, shared_den.at[pl.ds(rowd, RT_DEN)])

    pltpu.sync_copy(asrc_hbm, asv)
    pltpu.sync_copy(adst_hbm, adv)
    pltpu.sync_copy(m_hbm, mv)
    mvec = mv[...]
    plsc.subcore_barrier()

    base128 = s * (PER_W // 128)

    def _issue_idx(ch, sb, db):
        b128 = base128 + ch * NJ
        pltpu.async_copy(src_hbm.at[pl.ds(b128, NJ)], idx_s[sb], sem_i)
        pltpu.async_copy(dst_hbm.at[pl.ds(b128, NJ)], idx_d[db], sem_i)

    def _drain_idx(sb, db):
        pltpu.make_async_copy(src_hbm.at[pl.ds(base128, NJ)], idx_s[sb],
                              sem_i).wait()
        pltpu.make_async_copy(dst_hbm.at[pl.ds(base128, NJ)], idx_d[db],
                              sem_i).wait()

    def _fire_gathers(sb, rb):
        for j in range(NJ):
            @pl.when(c == 0)
            def _():
                pltpu.async_copy(ha_hbm.at[idx_s[sb].at[j]],
                                 rows[rb].at[pl.ds(j * 128, 128)], sem_g)

            @pl.when(c == 1)
            def _():
                pltpu.async_copy(hb_hbm.at[idx_s[sb].at[j]],
                                 rows[rb].at[pl.ds(j * 128, 128)], sem_g)

    def _drain_gathers(sb, rb):
        for j in range(NJ):
            pltpu.make_async_copy(ha_hbm.at[idx_s[sb].at[j]],
                                  rows[rb].at[pl.ds(j * 128, 128)],
                                  sem_g).wait()

    def _drain_scatters(rb, db):
        for j in range(NJ):
            pltpu.make_async_copy(rows[rb].at[pl.ds(j * 128, 128)],
                                  shared_out.at[idx_d[db].at[j]],
                                  sem_s).wait()
            pltpu.make_async_copy(wv[rb].at[pl.ds(j * 128, 128)],
                                  shared_den.at[idx_d[db].at[j]],
                                  sem_s).wait()

    def _chunk(t, g, k):
        rb, rbn = k % 2, (k + 1) % 2
        db, dbp = k % 3, (k + 2) % 3
        _drain_gathers(rb, rb)
        ebase = (base128 + g * NJ) * 128
        for i in range(C // 16):
            s16 = idx_s[rb][i // 8, pl.ds((i % 8) * 16, 16)]
            d16 = idx_d[db][i // 8, pl.ds((i % 8) * 16, 16)]
            e = plsc.load_gather(asv, [s16]) + plsc.load_gather(adv, [d16])
            e = jnp.where(e >= 0, e, e * 0.2)
            w = jnp.exp(e - mvec)
            eidx = ebase + i * 16 + lax.iota(jnp.int32, 16)
            w = jnp.where(eidx < E_TOT, w, 0.0)
            wv[rb][pl.ds(i * 16, 16)] = w
        _drain_idx(rbn, (k + 1) % 3)
        if k == 0:
            @pl.when(t >= 1)
            def _():
                _drain_scatters(rbn, dbp)
        else:
            _drain_scatters(rbn, dbp)
        _fire_gathers(rbn, rbn)
        _issue_idx(jnp.minimum(g + 2, CH - 1), rb, dbp)

        @plsc.parallel_loop(0, C // 16, unroll=2)
        def _srow(gg):
            w16 = wv[rb][pl.ds(gg * 16, 16)]
            for l in range(16):
                wf = jnp.full((16,), w16[l])
                r = gg * 16 + l
                vals = [rows[rb][r, pl.ds(j * 16, 16)] * wf
                        for j in range(DH // 16)]
                for j in range(DH // 16):
                    rows[rb][r, pl.ds(j * 16, 16)] = vals[j]

        for j in range(NJ):
            pltpu.async_copy(rows[rb].at[pl.ds(j * 128, 128)],
                             shared_out.at[idx_d[db].at[j]], sem_s, add=True)
            pltpu.async_copy(wv[rb].at[pl.ds(j * 128, 128)],
                             shared_den.at[idx_d[db].at[j]], sem_s, add=True)

    _issue_idx(0, 0, 0)
    _drain_idx(0, 0)
    _fire_gathers(0, 0)
    _issue_idx(1, 1, 1)

    def _step(t, _):
        for k in range(6):
            _chunk(t, 6 * t + k, k)
        return 0
    lax.fori_loop(0, CH // 6, _step, 0)

    _drain_scatters(1, (CH - 1) % 3)
    _drain_gathers(0, 0)
    _drain_idx(0, 0)

    plsc.subcore_barrier()

    offo = c * N_OUT + rowo
    ko = 0
    for k, ck in enumerate(OUT_CHUNKS):
        buf = rows[k % 2].at[pl.ds(0, ck)]
        pltpu.sync_copy(shared_out.at[pl.ds(rowo + ko, ck)], buf)
        pltpu.sync_copy(buf, outp_hbm.at[pl.ds(offo + ko, ck)])
        ko += ck
    pltpu.sync_copy(shared_den.at[pl.ds(rowd, RT_DEN)], z640)
    pltpu.sync_copy(z640, outd_hbm.at[pl.ds(c * N_DEN + rowd, RT_DEN)])


def kernel(x, edge_index, W, att_src, att_dst, bias):
    f32 = jnp.float32
    ha, hb, av, bv, m = pl.pallas_call(
        _tc_prep,
        out_shape=(
            jax.ShapeDtypeStruct((N, DH), f32),
            jax.ShapeDtypeStruct((N, DH), f32),
            jax.ShapeDtypeStruct((N, 1), f32),
            jax.ShapeDtypeStruct((N, 1), f32),
            jax.ShapeDtypeStruct((1, 1), f32),
        ),
    )(x, W, att_src.reshape(1, D), att_dst.reshape(1, D))

    loop = jnp.arange(N, dtype=jnp.int32)
    pad = jnp.zeros((E_PAD - E_TOT,), jnp.int32)
    src2d = jnp.concatenate([edge_index[0], loop, pad]).reshape(E_PAD // 128, 128)
    dst2d = jnp.concatenate([edge_index[1], loop, pad]).reshape(E_PAD // 128, 128)
    m16 = jnp.broadcast_to(m.reshape(1), (16,))

    sc = pl.kernel(
        _sc_edges,
        out_type=(
            jax.ShapeDtypeStruct((2 * N_OUT, DH), f32),
            jax.ShapeDtypeStruct((2 * N_DEN,), f32),
        ),
        mesh=plsc.VectorSubcoreMesh(core_axis_name="c", subcore_axis_name="s"),
        compiler_params=pltpu.CompilerParams(
            needs_layout_passes=False, use_tc_tiling_on_sc=False),
        scratch_types=(
            pltpu.VMEM((N,), f32),
            pltpu.VMEM((N,), f32),
            pltpu.VMEM((NJ, 128), jnp.int32),
            pltpu.VMEM((NJ, 128), jnp.int32),
            pltpu.VMEM((NJ, 128), jnp.int32),
            pltpu.VMEM((NJ, 128), jnp.int32),
            pltpu.VMEM((NJ, 128), jnp.int32),
            pltpu.VMEM((C,), f32),
            pltpu.VMEM((C,), f32),
            pltpu.VMEM((C, DH), f32),
            pltpu.VMEM((C, DH), f32),
            pltpu.VMEM((RT_DEN,), f32),
            pltpu.VMEM((16,), f32),
            pltpu.VMEM_SHARED((N_OUT, DH), f32),
            pltpu.VMEM_SHARED((N_DEN,), f32),
            pltpu.SemaphoreType.DMA,
            pltpu.SemaphoreType.DMA,
            pltpu.SemaphoreType.DMA,
        ),
    )
    outp, outd = sc(ha, hb, av.reshape(N), bv.reshape(N), src2d, dst2d, m16)

    p = outp.reshape(2, N_OUT, DH)
    d = outd.reshape(2, N_DEN)
    out = pl.pallas_call(
        _tc_finish,
        out_shape=jax.ShapeDtypeStruct((N, D), f32),
    )(p[0, :N], p[1, :N], d[0, :N, None], bias.reshape(1, D))
    return out


# speedup vs baseline: 1.1545x; 1.0273x over previous
import jax
import jax.numpy as jnp
from jax import lax
from jax.experimental import pallas as pl
from jax.experimental.pallas import tpu as pltpu
from jax.experimental.pallas import tpu_sc as plsc

N = 10000
D = 128
DH = D // 2
E = 320000
E_TOT = E + N
N_OUT = 10112
RT_OUT = N_OUT // 16
OUT_CHUNKS = (128, 128, 128, 128, 120)
N_DEN = 10240
RT_DEN = N_DEN // 16
C = 512
NJ = C // 128
CH = 42
PER_W = C * CH
E_PAD = PER_W * 16


def _tc_prep(x_ref, w_ref, as_ref, ad_ref, ha_ref, hb_ref, av_ref, bv_ref,
             m_ref):
    h = jnp.dot(x_ref[...], w_ref[...], preferred_element_type=jnp.float32)
    ha_ref[...] = h[:, :DH]
    hb_ref[...] = h[:, DH:]
    a1 = jnp.sum(h * as_ref[...], axis=1, keepdims=True)
    a2 = jnp.sum(h * ad_ref[...], axis=1, keepdims=True)
    av_ref[...] = a1
    bv_ref[...] = a2
    ms = jnp.max(a1) + jnp.max(a2)
    m_ref[...] = jnp.full((1, 1), jnp.where(ms >= 0, ms, ms * 0.2))


def _tc_finish(p0_ref, p1_ref, d_ref, b_ref, o_ref):
    den = d_ref[...]
    o_ref[...] = (jnp.concatenate([p0_ref[...], p1_ref[...]], axis=1) / den
                  + b_ref[...])


def _sc_edges(ha_hbm, hb_hbm, asrc_hbm, adst_hbm, src_hbm, dst_hbm, m_hbm,
              outp_hbm, outd_hbm,
              asv, adv, idx_s0, idx_s1, idx_d0, idx_d1, idx_d2,
              wv0, wv1, rows0, rows1, z640, mv,
              shared_out, shared_den, sem_i, sem_g, sem_s):
    c = lax.axis_index("c")
    s = lax.axis_index("s")
    zero16 = jnp.zeros((16,), jnp.float32)
    idx_s = (idx_s0, idx_s1)
    idx_d = (idx_d0, idx_d1, idx_d2)
    rows = (rows0, rows1)
    wv = (wv0, wv1)

    for i in range(RT_DEN // 16):
        z640[pl.ds(i * 16, 16)] = zero16

    def _zrow(i, _):
        for j in range(DH // 16):
            rows0[i, pl.ds(j * 16, 16)] = zero16
        return 0
    lax.fori_loop(0, 128, _zrow, 0)

    rowo = s * RT_OUT
    rowd = s * RT_DEN
    ko = 0
    for ck in OUT_CHUNKS:
        pltpu.sync_copy(rows0.at[pl.ds(0, ck)],
                        shared_out.at[pl.ds(rowo + ko, ck)])
        ko += ck
    pltpu.sync_copy(z640, shared_den.at[pl.ds(rowd, RT_DEN)])

    pltpu.sync_copy(asrc_hbm, asv)
    pltpu.sync_copy(adst_hbm, adv)
    pltpu.sync_copy(m_hbm, mv)
    mvec = mv[...]
    plsc.subcore_barrier()

    base128 = s * (PER_W // 128)

    def _issue_idx(ch, sb, db):
        b128 = base128 + ch * NJ
        pltpu.async_copy(src_hbm.at[pl.ds(b128, NJ)], idx_s[sb], sem_i)
        pltpu.async_copy(dst_hbm.at[pl.ds(b128, NJ)], idx_d[db], sem_i)

    def _drain_idx(sb, db):
        pltpu.make_async_copy(src_hbm.at[pl.ds(base128, NJ)], idx_s[sb],
                              sem_i).wait()
        pltpu.make_async_copy(dst_hbm.at[pl.ds(base128, NJ)], idx_d[db],
                              sem_i).wait()

    def _fire_gathers(sb, rb):
        for j in range(NJ):
            @pl.when(c == 0)
            def _():
                pltpu.async_copy(ha_hbm.at[idx_s[sb].at[j]],
                                 rows[rb].at[pl.ds(j * 128, 128)], sem_g)

            @pl.when(c == 1)
            def _():
                pltpu.async_copy(hb_hbm.at[idx_s[sb].at[j]],
                                 rows[rb].at[pl.ds(j * 128, 128)], sem_g)

    def _drain_gathers(sb, rb):
        for j in range(NJ):
            pltpu.make_async_copy(ha_hbm.at[idx_s[sb].at[j]],
                                  rows[rb].at[pl.ds(j * 128, 128)],
                                  sem_g).wait()

    def _drain_scatters(rb, db):
        for j in range(NJ):
            pltpu.make_async_copy(rows[rb].at[pl.ds(j * 128, 128)],
                                  shared_out.at[idx_d[db].at[j]],
                                  sem_s).wait()
            pltpu.make_async_copy(wv[rb].at[pl.ds(j * 128, 128)],
                                  shared_den.at[idx_d[db].at[j]],
                                  sem_s).wait()

    def _chunk(t, g, k):
        rb, rbn = k % 2, (k + 1) % 2
        db, dbp = k % 3, (k + 2) % 3
        _drain_gathers(rb, rb)
        _drain_idx(rbn, (k + 1) % 3)
        if k == 0:
            @pl.when(t >= 1)
            def _():
                _drain_scatters(rbn, dbp)
        else:
            _drain_scatters(rbn, dbp)
        _fire_gathers(rbn, rbn)
        ebase = (base128 + g * NJ) * 128

        @plsc.parallel_loop(0, C // 16, unroll=2)
        def _wloop(i):
            s16 = idx_s[rb][i // 8, pl.ds((i % 8) * 16, 16)]
            d16 = idx_d[db][i // 8, pl.ds((i % 8) * 16, 16)]
            e = plsc.load_gather(asv, [s16]) + plsc.load_gather(adv, [d16])
            e = jnp.where(e >= 0, e, e * 0.2)
            w = jnp.exp(e - mvec)
            eidx = ebase + i * 16 + lax.iota(jnp.int32, 16)
            w = jnp.where(eidx < E_TOT, w, 0.0)
            wv[rb][pl.ds(i * 16, 16)] = w
        _issue_idx(jnp.minimum(g + 2, CH - 1), rb, dbp)

        @plsc.parallel_loop(0, C // 16, unroll=2)
        def _srow(gg):
            w16 = wv[rb][pl.ds(gg * 16, 16)]
            for l in range(16):
                wf = jnp.full((16,), w16[l])
                r = gg * 16 + l
                vals = [rows[rb][r, pl.ds(j * 16, 16)] * wf
                        for j in range(DH // 16)]
                for j in range(DH // 16):
                    rows[rb][r, pl.ds(j * 16, 16)] = vals[j]

        for j in range(NJ):
            pltpu.async_copy(rows[rb].at[pl.ds(j * 128, 128)],
                             shared_out.at[idx_d[db].at[j]], sem_s, add=True)
            pltpu.async_copy(wv[rb].at[pl.ds(j * 128, 128)],
                             shared_den.at[idx_d[db].at[j]], sem_s, add=True)

    _issue_idx(0, 0, 0)
    _drain_idx(0, 0)
    _fire_gathers(0, 0)
    _issue_idx(1, 1, 1)

    def _step(t, _):
        for k in range(6):
            _chunk(t, 6 * t + k, k)
        return 0
    lax.fori_loop(0, CH // 6, _step, 0)

    _drain_scatters(1, (CH - 1) % 3)
    _drain_gathers(0, 0)
    _drain_idx(0, 0)

    plsc.subcore_barrier()

    offo = c * N_OUT + rowo
    ko = 0
    for k, ck in enumerate(OUT_CHUNKS):
        buf = rows[k % 2].at[pl.ds(0, ck)]
        pltpu.sync_copy(shared_out.at[pl.ds(rowo + ko, ck)], buf)
        pltpu.sync_copy(buf, outp_hbm.at[pl.ds(offo + ko, ck)])
        ko += ck
    pltpu.sync_copy(shared_den.at[pl.ds(rowd, RT_DEN)], z640)
    pltpu.sync_copy(z640, outd_hbm.at[pl.ds(c * N_DEN + rowd, RT_DEN)])


def kernel(x, edge_index, W, att_src, att_dst, bias):
    f32 = jnp.float32
    ha, hb, av, bv, m = pl.pallas_call(
        _tc_prep,
        out_shape=(
            jax.ShapeDtypeStruct((N, DH), f32),
            jax.ShapeDtypeStruct((N, DH), f32),
            jax.ShapeDtypeStruct((N, 1), f32),
            jax.ShapeDtypeStruct((N, 1), f32),
            jax.ShapeDtypeStruct((1, 1), f32),
        ),
    )(x, W, att_src.reshape(1, D), att_dst.reshape(1, D))

    loop = jnp.arange(N, dtype=jnp.int32)
    pad = jnp.zeros((E_PAD - E_TOT,), jnp.int32)
    src2d = jnp.concatenate([edge_index[0], loop, pad]).reshape(E_PAD // 128, 128)
    dst2d = jnp.concatenate([edge_index[1], loop, pad]).reshape(E_PAD // 128, 128)
    m16 = jnp.broadcast_to(m.reshape(1), (16,))

    sc = pl.kernel(
        _sc_edges,
        out_type=(
            jax.ShapeDtypeStruct((2 * N_OUT, DH), f32),
            jax.ShapeDtypeStruct((2 * N_DEN,), f32),
        ),
        mesh=plsc.VectorSubcoreMesh(core_axis_name="c", subcore_axis_name="s"),
        compiler_params=pltpu.CompilerParams(
            needs_layout_passes=False, use_tc_tiling_on_sc=False),
        scratch_types=(
            pltpu.VMEM((N,), f32),
            pltpu.VMEM((N,), f32),
            pltpu.VMEM((NJ, 128), jnp.int32),
            pltpu.VMEM((NJ, 128), jnp.int32),
            pltpu.VMEM((NJ, 128), jnp.int32),
            pltpu.VMEM((NJ, 128), jnp.int32),
            pltpu.VMEM((NJ, 128), jnp.int32),
            pltpu.VMEM((C,), f32),
            pltpu.VMEM((C,), f32),
            pltpu.VMEM((C, DH), f32),
            pltpu.VMEM((C, DH), f32),
            pltpu.VMEM((RT_DEN,), f32),
            pltpu.VMEM((16,), f32),
            pltpu.VMEM_SHARED((N_OUT, DH), f32),
            pltpu.VMEM_SHARED((N_DEN,), f32),
            pltpu.SemaphoreType.DMA,
            pltpu.SemaphoreType.DMA,
            pltpu.SemaphoreType.DMA,
        ),
    )
    outp, outd = sc(ha, hb, av.reshape(N), bv.reshape(N), src2d, dst2d, m16)

    p = outp.reshape(2, N_OUT, DH)
    d = outd.reshape(2, N_DEN)
    out = pl.pallas_call(
        _tc_finish,
        out_shape=jax.ShapeDtypeStruct((N, D), f32),
    )(p[0, :N], p[1, :N], d[0, :N, None], bias.reshape(1, D))
    return out
